# baseline (speedup 1.0000x reference)
import jax
import jax.numpy as jnp
from jax.experimental import pallas as pl
from jax.experimental.pallas import tpu as pltpu

_HEADS = 32
_PAIRS = 16
_D = 128
_TB = 512
_TC = 16


def _fwht16(vs):
    for b in (8, 4, 2, 1):
        nv = [None] * _PAIRS
        for q in range(0, _PAIRS, 2 * b):
            for r in range(b):
                i0, i1 = q + r, q + r + b
                nv[i0] = vs[i0] + vs[i1]
                nv[i1] = vs[i0] - vs[i1]
        vs = nv
    return vs


def _body(x_ref, w_hbm, o_ref, u_ref, w_ref, w_sem):
    nchunks = _TB // _TC

    @pl.when(pl.program_id(1) == 0)
    def _():
        pltpu.make_async_copy(w_hbm, w_ref, w_sem).start()
        pltpu.make_async_copy(w_hbm, w_ref, w_sem).wait()

    def mix1(i, carry):
        rows = pl.ds(i * _TC, _TC)
        vs = [
            jnp.concatenate(
                [x_ref[0, 2 * p, rows, :], x_ref[0, 2 * p + 1, rows, :]],
                axis=-1,
            )
            for p in range(_PAIRS)
        ]
        vs = _fwht16(vs)
        for p in range(_PAIRS):
            u_ref[p, rows, :] = vs[p].astype(jnp.bfloat16)
        return carry

    jax.lax.fori_loop(0, nchunks, mix1, 0)

    for p in range(_PAIRS):
        z = jnp.dot(u_ref[p, :, :], w_ref[p], preferred_element_type=jnp.float32)
        o_ref[0, 2 * p, :, :] = z[:, :_D]
        o_ref[0, 2 * p + 1, :, :] = z[:, _D:]

    def mix2(i, carry):
        rows = pl.ds(i * _TC, _TC)
        vs = _fwht16(
            [
                jnp.concatenate(
                    [o_ref[0, 2 * p, rows, :], o_ref[0, 2 * p + 1, rows, :]],
                    axis=-1,
                )
                for p in range(_PAIRS)
            ]
        )
        for p in range(_PAIRS):
            o_ref[0, 2 * p, rows, :] = vs[p][:, :_D]
            o_ref[0, 2 * p + 1, rows, :] = vs[p][:, _D:]
        return carry

    jax.lax.fori_loop(0, nchunks, mix2, 0)


def kernel(x, W, beta):
    B, H, T, D = x.shape
    Wa, Wb = W[0::2], W[1::2]
    S, Dm = Wa + Wb, Wa - Wb
    top = jnp.concatenate([S, Dm], axis=-1)
    bot = jnp.concatenate([Dm, S], axis=-1)
    Wp = jnp.concatenate([top, bot], axis=-2)
    scale = jnp.concatenate([beta, beta]) * (1.0 / _HEADS)
    Wp = (Wp * scale[None, None, :]).astype(jnp.bfloat16)

    return pl.pallas_call(
        _body,
        grid=(B, T // _TB),
        in_specs=[
            pl.BlockSpec((1, H, _TB, D), lambda b, t: (b, 0, t, 0)),
            pl.BlockSpec(memory_space=pl.ANY),
        ],
        out_specs=pl.BlockSpec((1, H, _TB, D), lambda b, t: (b, 0, t, 0)),
        out_shape=jax.ShapeDtypeStruct(x.shape, x.dtype),
        scratch_shapes=[
            pltpu.VMEM((_PAIRS, _TB, 2 * D), jnp.bfloat16),
            pltpu.VMEM((_PAIRS, 2 * D, 2 * D), jnp.bfloat16),
            pltpu.SemaphoreType.DMA,
        ],
        compiler_params=pltpu.CompilerParams(
            dimension_semantics=("parallel", "parallel"),
        ),
    )(x, Wp)


# speedup vs baseline: 1.1077x; 1.1077x over previous
import jax
import jax.numpy as jnp
from jax.experimental import pallas as pl
from jax.experimental.pallas import tpu as pltpu

_HEADS = 32
_PAIRS = 16
_D = 128
_TB = 512
_TC = 16


def _fwht16(vs):
    for b in (8, 4, 2, 1):
        nv = [None] * _PAIRS
        for q in range(0, _PAIRS, 2 * b):
            for r in range(b):
                i0, i1 = q + r, q + r + b
                nv[i0] = vs[i0] + vs[i1]
                nv[i1] = vs[i0] - vs[i1]
        vs = nv
    return vs


def _body(x_ref, w_ref, o_ref, u_ref):
    nchunks = _TB // _TC

    def mix1(i, carry):
        rows = pl.ds(i * _TC, _TC)
        vs = [
            jnp.concatenate(
                [x_ref[0, 2 * p, rows, :], x_ref[0, 2 * p + 1, rows, :]],
                axis=-1,
            )
            for p in range(_PAIRS)
        ]
        vs = _fwht16(vs)
        for p in range(_PAIRS):
            u_ref[p, rows, :] = vs[p].astype(jnp.bfloat16)
        return carry

    jax.lax.fori_loop(0, nchunks, mix1, 0)

    for p in range(_PAIRS):
        z = jnp.dot(u_ref[p, :, :], w_ref[p], preferred_element_type=jnp.float32)
        o_ref[0, 2 * p, :, :] = z[:, :_D]
        o_ref[0, 2 * p + 1, :, :] = z[:, _D:]

    def mix2(i, carry):
        rows = pl.ds(i * _TC, _TC)
        vs = _fwht16(
            [
                jnp.concatenate(
                    [o_ref[0, 2 * p, rows, :], o_ref[0, 2 * p + 1, rows, :]],
                    axis=-1,
                )
                for p in range(_PAIRS)
            ]
        )
        for p in range(_PAIRS):
            o_ref[0, 2 * p, rows, :] = vs[p][:, :_D]
            o_ref[0, 2 * p + 1, rows, :] = vs[p][:, _D:]
        return carry

    jax.lax.fori_loop(0, nchunks, mix2, 0)


def kernel(x, W, beta):
    B, H, T, D = x.shape
    Wa, Wb = W[0::2], W[1::2]
    S, Dm = Wa + Wb, Wa - Wb
    top = jnp.concatenate([S, Dm], axis=-1)
    bot = jnp.concatenate([Dm, S], axis=-1)
    Wp = jnp.concatenate([top, bot], axis=-2)
    scale = jnp.concatenate([beta, beta]) * (1.0 / _HEADS)
    Wp = (Wp * scale[None, None, :]).astype(jnp.bfloat16)

    return pl.pallas_call(
        _body,
        grid=(B, T // _TB),
        in_specs=[
            pl.BlockSpec((1, H, _TB, D), lambda b, t: (b, 0, t, 0)),
            pl.BlockSpec((_PAIRS, 2 * D, 2 * D), lambda b, t: (0, 0, 0)),
        ],
        out_specs=pl.BlockSpec((1, H, _TB, D), lambda b, t: (b, 0, t, 0)),
        out_shape=jax.ShapeDtypeStruct(x.shape, x.dtype),
        scratch_shapes=[pltpu.VMEM((_PAIRS, _TB, 2 * D), jnp.bfloat16)],
        compiler_params=pltpu.CompilerParams(
            dimension_semantics=("parallel", "parallel"),
        ),
    )(x, Wp)
